# baseline (speedup 1.0000x reference)
import jax
import jax.numpy as jnp
from jax import lax
from jax.experimental import pallas as pl
from jax.experimental.pallas import tpu as pltpu
from jax.experimental.pallas import tpu_sc as plsc

B, P, D = 4, 8192, 768
N = B * P
LANES = 16
NC, NS = 2, 16
NW = NC * NS
RPW = N // NW
C = 8
NCHUNK = RPW // C
GROUPS = D // LANES
NBUF = 4


def _pe_add_kernel(x_hbm, idx_hbm, pe_hbm, out_hbm, idx_v, xsh, *scratch):
    xbufs = scratch[0:NBUF]
    pebufs = scratch[NBUF:2 * NBUF]
    sem_x = scratch[2 * NBUF:3 * NBUF]
    sem_pe = scratch[3 * NBUF:4 * NBUF]
    sem_out = scratch[4 * NBUF:5 * NBUF]
    sem_h1 = scratch[5 * NBUF:6 * NBUF]

    s = lax.axis_index("s")
    wid = s * NC + lax.axis_index("c")
    base = wid * RPW
    pltpu.sync_copy(idx_hbm.at[pl.ds(base, RPW)], idx_v)

    def start_h1(i, sp):
        pltpu.make_async_copy(
            x_hbm.at[pl.ds(base + i * C, C)], xsh.at[s, sp], sem_h1[sp]).start()

    def wait_h1(i, sp):
        pltpu.make_async_copy(
            x_hbm.at[pl.ds(base + i * C, C)], xsh.at[s, sp], sem_h1[sp]).wait()

    def start_h2(i, b):
        pltpu.make_async_copy(xsh.at[s, b], xbufs[b], sem_x[b]).start()

    def wait_h2(i, b):
        pltpu.make_async_copy(xsh.at[s, b], xbufs[b], sem_x[b]).wait()

    def start_pe(i, b):
        pltpu.make_async_copy(
            pe_hbm.at[idx_v.at[pl.ds(i * C, C)]], pebufs[b], sem_pe[b]).start()

    def wait_pe(i, b):
        pltpu.make_async_copy(
            pe_hbm.at[idx_v.at[pl.ds(i * C, C)]], pebufs[b], sem_pe[b]).wait()

    def start_out(i, b):
        pltpu.make_async_copy(
            xbufs[b], out_hbm.at[pl.ds(base + i * C, C)], sem_out[b]).start()

    def wait_out(i, b):
        pltpu.make_async_copy(
            xbufs[b], out_hbm.at[pl.ds(base + i * C, C)], sem_out[b]).wait()

    def body(i, b, traced):
        nb = (b + 2) % NBUF

        if traced:
            @pl.when(i >= 2)
            def _():
                wait_out(i - 2, nb)

            @pl.when(i + 2 < NCHUNK)
            def _():
                wait_h1(i + 2, nb)
                start_h2(i + 2, nb)
                start_pe(i + 2, nb)
        else:
            if i >= 2:
                wait_out(i - 2, nb)
            if i + 2 < NCHUNK:
                wait_h1(i + 2, nb)
                start_h2(i + 2, nb)
                start_pe(i + 2, nb)

        wait_h2(i, b)
        wait_pe(i, b)

        if traced:
            @pl.when(i + 4 < NCHUNK)
            def _():
                start_h1(i + 4, b)
        else:
            if i + 4 < NCHUNK:
                start_h1(i + 4, b)

        def row_body(r, _):
            for k in range(GROUPS):
                plsc.addupdate(xbufs[b].at[r, pl.ds(k * LANES, LANES)],
                               pebufs[b][r, pl.ds(k * LANES, LANES)])
            return 0

        lax.fori_loop(0, C, row_body, 0)
        start_out(i, b)

    for i in range(NBUF):
        start_h1(i, i)
    for i in range(2):
        wait_h1(i, i)
        start_h2(i, i)
        start_pe(i, i)

    def outer(i0, _):
        for b in range(NBUF):
            body(i0 + b, b, traced=True)
        return 0

    lax.fori_loop(0, NCHUNK // NBUF, lambda t, c: outer(t * NBUF, c), 0)

    for i in range(NCHUNK - 2, NCHUNK):
        wait_out(i, i % NBUF)


@jax.jit
def kernel(x, indices, pe):
    x2 = x.reshape(N, D)
    idx = indices.reshape(N)
    tab = pe.reshape(P, D)
    mesh = plsc.VectorSubcoreMesh(core_axis_name="c", subcore_axis_name="s")
    out = pl.kernel(
        _pe_add_kernel,
        out_type=jax.ShapeDtypeStruct((N, D), jnp.float32),
        mesh=mesh,
        scratch_types=(
            [pltpu.VMEM((RPW,), jnp.int32)]
            + [pltpu.VMEM_SHARED((NS, NBUF, C, D), jnp.float32)]
            + [pltpu.VMEM((C, D), jnp.float32) for _ in range(NBUF)]
            + [pltpu.VMEM((C, D), jnp.float32) for _ in range(NBUF)]
            + [pltpu.SemaphoreType.DMA for _ in range(4 * NBUF)]
        ),
    )(x2, idx, tab)
    return out.reshape(B, P, D)
